# Initial kernel scaffold; baseline (speedup 1.0000x reference)
import jax
import jax.numpy as jnp
from jax.experimental import pallas as pl


def kernel(x, edge_index, W1_l, b1_l, W1_r, W2_l, b2_l, W2_r):
    raise NotImplementedError("write your pallas kernel here")


# speedup vs baseline: 3.3704x; 3.3704x over previous
import functools

import jax
import jax.numpy as jnp
from jax import lax
from jax.experimental import pallas as pl
from jax.experimental.pallas import tpu as pltpu
from jax.experimental.pallas import tpu_sc as plsc

N = 10000
D = 128
E = 320000
NC, NS = 2, 16
DH = D // NC
EPW = E // NS
C = 80
K = EPW // C
NP = 10240
STRIPE = NP // NS
CW = 8

_mesh = plsc.VectorSubcoreMesh(
    core_axis_name="c", subcore_axis_name="s", num_cores=NC, num_subcores=NS
)


@functools.partial(
    pl.kernel,
    out_type=(
        jax.ShapeDtypeStruct((NC, NP, DH), jnp.float32),
        jax.ShapeDtypeStruct((NC, NP, CW), jnp.float32),
    ),
    mesh=_mesh,
    compiler_params=pltpu.CompilerParams(use_tc_tiling_on_sc=False),
    scratch_types=[
        pltpu.VMEM_SHARED((NP, DH), jnp.float32),
        pltpu.VMEM_SHARED((NP, CW), jnp.float32),
        pltpu.VMEM((C,), jnp.int32),
        pltpu.VMEM((C,), jnp.int32),
        pltpu.VMEM((C, DH), jnp.float32),
        pltpu.VMEM((C, CW), jnp.float32),
        pltpu.VMEM((STRIPE, DH), jnp.float32),
        pltpu.VMEM((STRIPE, CW), jnp.float32),
        pltpu.SemaphoreType.DMA,
    ],
)
def _agg_pass(t_hbm, src_hbm, dst_hbm, zeros_hbm, z8_hbm, ones_hbm,
              aggp_hbm, cntp_hbm,
              acc_sh, cnt_sh, idx_s, idx_d, rows, ones_v, tmp, tmp8, sem):
    cid = lax.axis_index("c")
    sid = lax.axis_index("s")
    rbase = sid * STRIPE
    pltpu.sync_copy(zeros_hbm, tmp)
    pltpu.sync_copy(tmp, acc_sh.at[pl.ds(rbase, STRIPE)])
    pltpu.sync_copy(z8_hbm, tmp8)
    pltpu.sync_copy(tmp8, cnt_sh.at[pl.ds(rbase, STRIPE)])
    pltpu.sync_copy(ones_hbm, ones_v)
    plsc.subcore_barrier()

    ebase = sid * EPW

    def step(k, carry):
        base = ebase + k * C
        pltpu.sync_copy(src_hbm.at[pl.ds(base, C)], idx_s)
        pltpu.async_copy(t_hbm.at[cid].at[idx_s], rows, sem).wait()
        pltpu.sync_copy(dst_hbm.at[pl.ds(base, C)], idx_d)
        pltpu.sync_copy(rows, acc_sh.at[idx_d], add=True)
        pltpu.sync_copy(ones_v, cnt_sh.at[idx_d], add=True)
        return carry

    lax.fori_loop(0, K, step, 0)
    plsc.subcore_barrier()
    pltpu.sync_copy(acc_sh.at[pl.ds(rbase, STRIPE)], tmp)
    pltpu.sync_copy(tmp, aggp_hbm.at[cid].at[pl.ds(rbase, STRIPE)])
    pltpu.sync_copy(cnt_sh.at[pl.ds(rbase, STRIPE)], tmp8)
    pltpu.sync_copy(tmp8, cntp_hbm.at[cid].at[pl.ds(rbase, STRIPE)])


def _dotT(a, w):
    return lax.dot_general(a, w, (((1,), (1,)), ((), ())),
                           preferred_element_type=jnp.float32)


def _split_cols(t_ref, res):
    for c in range(NC):
        t_ref[c] = res[:, c * DH:(c + 1) * DH]


def _lin2_body(x_ref, wl_ref, wr_ref, b_ref, t_ref, r_ref):
    x = x_ref[...]
    _split_cols(t_ref, _dotT(x, wl_ref[...]))
    r_ref[...] = _dotT(x, wr_ref[...]) + b_ref[...][None, :]


def _lin2(x, wl, wr, b):
    return pl.pallas_call(
        _lin2_body,
        out_shape=(
            jax.ShapeDtypeStruct((NC, N, DH), jnp.float32),
            jax.ShapeDtypeStruct((N, D), jnp.float32),
        ),
    )(x, wl, wr, b)


def _mean(p_ref, cntp_ref):
    cnt = cntp_ref[0, 0:N, 0:1]
    inv = 1.0 / jnp.maximum(cnt, 1.0)
    agg = jnp.concatenate([p_ref[0, 0:N, :], p_ref[1, 0:N, :]], axis=1)
    return agg * inv


def _mid_body(p_ref, cntp_ref, r1_ref, wl_ref, wr_ref, b_ref, t_ref, r_ref):
    h = jnp.maximum(_mean(p_ref, cntp_ref) + r1_ref[...], 0.0)
    _split_cols(t_ref, _dotT(h, wl_ref[...]))
    r_ref[...] = _dotT(h, wr_ref[...]) + b_ref[...][None, :]


def _mid(aggp, cntp, r1, wl, wr, b):
    return pl.pallas_call(
        _mid_body,
        out_shape=(
            jax.ShapeDtypeStruct((NC, N, DH), jnp.float32),
            jax.ShapeDtypeStruct((N, D), jnp.float32),
        ),
    )(aggp, cntp, r1, wl, wr, b)


def _final_body(q_ref, cntp_ref, r2_ref, o_ref):
    o_ref[...] = _mean(q_ref, cntp_ref) + r2_ref[...]


def _final(qp, cntp, r2):
    return pl.pallas_call(
        _final_body,
        out_shape=jax.ShapeDtypeStruct((N, D), jnp.float32),
    )(qp, cntp, r2)


def kernel(x, edge_index, W1_l, b1_l, W1_r, W2_l, b2_l, W2_r):
    src = edge_index[0].astype(jnp.int32)
    dst = edge_index[1].astype(jnp.int32)
    zeros = jnp.zeros((STRIPE, DH), jnp.float32)
    z8 = jnp.zeros((STRIPE, CW), jnp.float32)
    ones = jnp.ones((C, CW), jnp.float32)
    t1, r1 = _lin2(x, W1_l, W1_r, b1_l)
    aggp, cntp = _agg_pass(t1, src, dst, zeros, z8, ones)
    t2, r2 = _mid(aggp, cntp, r1, W2_l, W2_r, b2_l)
    qp, _ = _agg_pass(t2, src, dst, zeros, z8, ones)
    return _final(qp, cntp, r2)


# speedup vs baseline: 9.1772x; 2.7229x over previous
import functools

import jax
import jax.numpy as jnp
from jax import lax
from jax.experimental import pallas as pl
from jax.experimental.pallas import tpu as pltpu
from jax.experimental.pallas import tpu_sc as plsc

N = 10000
D = 128
E = 320000
NC, NS = 2, 16
DH = D // NC
EPW = E // NS
C = 80
K = EPW // C
KH = K // 2
NP = 10240
STRIPE = NP // NS
CW = 8

_mesh = plsc.VectorSubcoreMesh(
    core_axis_name="c", subcore_axis_name="s", num_cores=NC, num_subcores=NS
)


@functools.partial(
    pl.kernel,
    out_type=(
        jax.ShapeDtypeStruct((NC, NP, DH), jnp.float32),
        jax.ShapeDtypeStruct((NC, NP, CW), jnp.float32),
    ),
    mesh=_mesh,
    compiler_params=pltpu.CompilerParams(use_tc_tiling_on_sc=False),
    scratch_types=[
        pltpu.VMEM_SHARED((NP, DH), jnp.float32),
        pltpu.VMEM_SHARED((NP, CW), jnp.float32),
        pltpu.VMEM((K, C), jnp.int32),
        pltpu.VMEM((K, C), jnp.int32),
        pltpu.VMEM((C, DH), jnp.float32),
        pltpu.VMEM((C, DH), jnp.float32),
        pltpu.VMEM((C, CW), jnp.float32),
        pltpu.SemaphoreType.DMA,
        pltpu.SemaphoreType.DMA,
    ],
)
def _agg_pass(t_hbm, srcg_hbm, dstr_hbm, zeros_hbm, z8_hbm, ones_hbm,
              aggp_hbm, cntp_hbm,
              acc_sh, cnt_sh, idxs_v, idxd_v, rows0, rows1, ones_v,
              sem0, sem1):
    cid = lax.axis_index("c")
    sid = lax.axis_index("s")
    rbase = sid * STRIPE
    pltpu.sync_copy(srcg_hbm.at[cid].at[sid], idxs_v)
    pltpu.sync_copy(dstr_hbm.at[sid], idxd_v)
    pltpu.sync_copy(ones_hbm, ones_v)
    pltpu.sync_copy(zeros_hbm, acc_sh.at[pl.ds(rbase, STRIPE)])
    pltpu.sync_copy(z8_hbm, cnt_sh.at[pl.ds(rbase, STRIPE)])
    plsc.subcore_barrier()

    def gather(k, rows, sem):
        pltpu.async_copy(t_hbm.at[idxs_v.at[k]], rows, sem)

    def gwait(k, rows, sem):
        pltpu.make_async_copy(t_hbm.at[idxs_v.at[k]], rows, sem).wait()

    def put(k, rows):
        pltpu.sync_copy(rows, acc_sh.at[idxd_v.at[k]], add=True)
        do_cnt = jnp.where(cid == 0, k < KH, k >= KH)
        @pl.when(do_cnt)
        def _():
            pltpu.sync_copy(ones_v, cnt_sh.at[idxd_v.at[k]], add=True)

    gather(0, rows0, sem0)

    def step(i, carry):
        k = 2 * i
        gather(k + 1, rows1, sem1)
        gwait(k, rows0, sem0)
        put(k, rows0)

        @pl.when(i < K // 2 - 1)
        def _():
            gather(k + 2, rows0, sem0)
        gwait(k + 1, rows1, sem1)
        put(k + 1, rows1)
        return carry

    lax.fori_loop(0, K // 2, step, 0)
    plsc.subcore_barrier()
    pltpu.sync_copy(acc_sh.at[pl.ds(rbase, STRIPE)], aggp_hbm.at[cid].at[pl.ds(rbase, STRIPE)])
    pltpu.sync_copy(cnt_sh.at[pl.ds(rbase, STRIPE)], cntp_hbm.at[cid].at[pl.ds(rbase, STRIPE)])


def _dotT(a, w):
    return lax.dot_general(a, w, (((1,), (1,)), ((), ())),
                           preferred_element_type=jnp.float32)


def _lin2_body(x_ref, wl_ref, wr_ref, b_ref, t_ref, r_ref):
    x = x_ref[...]
    t_ref[...] = _dotT(x, wl_ref[...])
    r_ref[...] = _dotT(x, wr_ref[...]) + b_ref[...][None, :]


def _lin2(x, wl, wr, b):
    return pl.pallas_call(
        _lin2_body,
        out_shape=(
            jax.ShapeDtypeStruct((N, D), jnp.float32),
            jax.ShapeDtypeStruct((N, D), jnp.float32),
        ),
    )(x, wl, wr, b)


def _mean(p_ref, cntp_ref):
    cnt = cntp_ref[0, 0:N, 0:1] + cntp_ref[1, 0:N, 0:1]
    inv = 1.0 / jnp.maximum(cnt, 1.0)
    agg = jnp.concatenate([p_ref[0, 0:N, :], p_ref[1, 0:N, :]], axis=1)
    return agg * inv


def _mid_body(p_ref, cntp_ref, r1_ref, wl_ref, wr_ref, b_ref, t_ref, r_ref):
    h = jnp.maximum(_mean(p_ref, cntp_ref) + r1_ref[...], 0.0)
    t_ref[...] = _dotT(h, wl_ref[...])
    r_ref[...] = _dotT(h, wr_ref[...]) + b_ref[...][None, :]


def _mid(aggp, cntp, r1, wl, wr, b):
    return pl.pallas_call(
        _mid_body,
        out_shape=(
            jax.ShapeDtypeStruct((N, D), jnp.float32),
            jax.ShapeDtypeStruct((N, D), jnp.float32),
        ),
    )(aggp, cntp, r1, wl, wr, b)


def _final_body(q_ref, cntp_ref, r2_ref, o_ref):
    o_ref[...] = _mean(q_ref, cntp_ref) + r2_ref[...]


def _final(qp, cntp, r2):
    return pl.pallas_call(
        _final_body,
        out_shape=jax.ShapeDtypeStruct((N, D), jnp.float32),
    )(qp, cntp, r2)


def kernel(x, edge_index, W1_l, b1_l, W1_r, W2_l, b2_l, W2_r):
    src = edge_index[0].astype(jnp.int32)
    dst = edge_index[1].astype(jnp.int32)
    srcg = (2 * src)[None, :] + jnp.arange(NC, dtype=jnp.int32)[:, None]
    srcg = srcg.reshape(NC, NS, K, C)
    dstr = dst.reshape(NS, K, C)
    zeros = jnp.zeros((STRIPE, DH), jnp.float32)
    z8 = jnp.zeros((STRIPE, CW), jnp.float32)
    ones = jnp.ones((C, CW), jnp.float32)
    t1, r1 = _lin2(x, W1_l, W1_r, b1_l)
    aggp, cntp = _agg_pass(t1.reshape(2 * N, DH), srcg, dstr, zeros, z8, ones)
    t2, r2 = _mid(aggp, cntp, r1, W2_l, W2_r, b2_l)
    qp, _ = _agg_pass(t2.reshape(2 * N, DH), srcg, dstr, zeros, z8, ones)
    return _final(qp, cntp, r2)
